# baseline (speedup 1.0000x reference)
import numpy as np
import jax
import jax.numpy as jnp
from jax import lax
from jax.experimental import pallas as pl
from jax.experimental.pallas import tpu as pltpu
from jax.experimental.pallas import tpu_sc as plsc

MAX_ATOMIC_NUM = 100
NUM_NOISE_LEVEL = 50
N_CRYSTALS = 4096
ATOMS_PER_CRYSTAL = 32
N_ATOMS = N_CRYSTALS * ATOMS_PER_CRYSTAL

CR_PER_BLK = 256
A_PER_BLK = CR_PER_BLK * ATOMS_PER_CRYSTAL

C_SC = 1536
A_SC = C_SC * ATOMS_PER_CRYSTAL
NW = 32
W_PER = A_SC * 128 // NW
SC_CHUNK = 16384
SC_NCHUNK = W_PER // SC_CHUNK

_U32 = np.uint32


def _np_threefry2x32(k0, k1, x0, x1):
    ks = [_U32(k0), _U32(k1), _U32(k0) ^ _U32(k1) ^ _U32(0x1BD11BDA)]
    rot = ([13, 15, 26, 6], [17, 29, 16, 24])
    x0 = np.asarray(x0, _U32)
    x1 = np.asarray(x1, _U32)
    with np.errstate(over="ignore"):
        x0 = x0 + ks[0]
        x1 = x1 + ks[1]
        for i in range(5):
            for r in rot[i % 2]:
                x0 = x0 + x1
                x1 = ((x1 << _U32(r)) | (x1 >> _U32(32 - r))) ^ x0
            x0 = x0 + ks[(i + 1) % 3]
            x1 = x1 + ks[(i + 2) % 3] + _U32(i + 1)
    return x0, x1


def _np_split(key, num):
    o0, o1 = _np_threefry2x32(key[0], key[1], np.zeros(num, _U32),
                              np.arange(num, dtype=_U32))
    return [(int(o0[i]), int(o1[i])) for i in range(num)]


_K1, _K2, _K3, _K4 = _np_split((0, 42), 4)
_K1A, _K1B = _np_split(_K1, 2)
_K2A, _K2B = _np_split(_K2, 2)

_TINY = np.float32(np.finfo(np.float32).tiny)
_NEXT_M1 = np.float32(np.nextafter(np.float32(-1.0), np.float32(0.0)))
_SPAN2 = np.float32(np.float32(1.0) - _NEXT_M1)
_SPAN1 = np.float32(np.float32(1.0) - _TINY)
_SQRT2 = np.float32(np.sqrt(2.0))


def _tf_bits(key, x1):
    k0, k1 = key
    ks = [k0 & 0xFFFFFFFF, k1 & 0xFFFFFFFF, (k0 ^ k1 ^ 0x1BD11BDA) & 0xFFFFFFFF]
    rot = ([13, 15, 26, 6], [17, 29, 16, 24])
    x0 = jnp.uint32(ks[0])
    x1 = x1 + jnp.uint32(ks[1])
    for i in range(5):
        for r in rot[i % 2]:
            x0 = x0 + x1
            x1 = lax.shift_left(x1, jnp.uint32(r)) | lax.shift_right_logical(
                x1, jnp.uint32(32 - r))
            x1 = x0 ^ x1
        x0 = x0 + jnp.uint32(ks[(i + 1) % 3])
        x1 = x1 + jnp.uint32((ks[(i + 2) % 3] + i + 1) & 0xFFFFFFFF)
    return x0 ^ x1


def _unit_float(bits):
    fb = lax.shift_right_logical(bits, jnp.uint32(9)) | jnp.uint32(0x3F800000)
    return lax.bitcast_convert_type(fb, jnp.float32) - jnp.float32(1.0)


def _mod50(u):
    a = lax.shift_right_logical(u, jnp.uint32(16))
    b = u & jnp.uint32(0xFFFF)
    m = (a * jnp.uint32(36) + b).astype(jnp.int32)
    q = jnp.floor(m.astype(jnp.float32) * jnp.float32(0.02)).astype(jnp.int32)
    r = m - q * jnp.int32(50)
    r = jnp.where(r < 0, r + jnp.int32(50), r)
    r = jnp.where(r >= jnp.int32(50), r - jnp.int32(50), r)
    r = jnp.where(r >= jnp.int32(50), r - jnp.int32(50), r)
    return r


def _erfinv(x):
    w = -jnp.log((jnp.float32(1.0) - x) * (jnp.float32(1.0) + x))
    ws = w - jnp.float32(2.5)
    wb = jnp.sqrt(w) - jnp.float32(3.0)
    cs = [2.81022636e-08, 3.43273939e-07, -3.5233877e-06, -4.39150654e-06,
          0.00021858087, -0.00125372503, -0.00417768164, 0.246640727,
          1.50140941]
    cb = [-0.000200214257, 0.000100950558, 0.00134934322, -0.00367342844,
          0.00573950773, -0.0076224613, 0.00943887047, 1.00167406, 2.83297682]
    ps = jnp.float32(cs[0])
    for c in cs[1:]:
        ps = jnp.float32(c) + ps * ws
    pb = jnp.float32(cb[0])
    for c in cb[1:]:
        pb = jnp.float32(c) + pb * wb
    return jnp.where(w < jnp.float32(5.0), ps, pb) * x


def _levels_kernel(sig_ref, tsig_ref, sx_ref, st_ref):
    i = (jnp.uint32(128) * lax.broadcasted_iota(jnp.uint32, (32, 128), 0)
         + lax.broadcasted_iota(jnp.uint32, (32, 128), 1))
    mult = jnp.int32(46)

    def levels(ka, kb):
        hi = _mod50(_tf_bits(ka, i))
        lo = _mod50(_tf_bits(kb, i))
        return _mod50((hi * mult + lo).astype(jnp.uint32))

    lvl_x = levels(_K1A, _K1B)
    lvl_t = levels(_K2A, _K2B)
    sx = jnp.zeros((32, 128), jnp.float32)
    st = jnp.zeros((32, 128), jnp.float32)
    for l in range(NUM_NOISE_LEVEL):
        sx = jnp.where(lvl_x == jnp.int32(l), sig_ref[l], sx)
        st = jnp.where(lvl_t == jnp.int32(l), tsig_ref[l], st)
    sx_ref[...] = sx
    st_ref[...] = st


def _sc_uniform_kernel(out_hbm, buf0, buf1, sem0, sem1):
    wid = lax.axis_index("s") * 2 + lax.axis_index("c")
    base = wid * W_PER
    lane = lax.iota(jnp.uint32, 16)
    bufs = (buf0, buf1)
    sems = (sem0, sem1)
    copies = []
    for step in range(SC_NCHUNK):
        buf = bufs[step % 2]
        if step >= 2:
            copies[step - 2].wait()
        cbase = lax.convert_element_type(base + step * SC_CHUNK, jnp.uint32)

        def body(j, _, buf=buf, cbase=cbase):
            q = cbase + lax.convert_element_type(j * 16, jnp.uint32) + lane
            c = (jnp.uint32(100) * lax.shift_right_logical(q, jnp.uint32(7))
                 + (q & jnp.uint32(127)))
            buf[pl.ds(j * 16, 16)] = _unit_float(_tf_bits(_K4, c))
            return 0

        lax.fori_loop(0, SC_CHUNK // 16, body, 0)
        cp = pltpu.make_async_copy(
            buf, out_hbm.at[pl.ds(base + step * SC_CHUNK, SC_CHUNK)],
            sems[step % 2])
        cp.start()
        copies.append(cp)
    copies[SC_NCHUNK - 2].wait()
    copies[SC_NCHUNK - 1].wait()


def _sigma_row(scr, stc_ref, cr0):
    for c in range(CR_PER_BLK):
        scr[:, pl.ds(c * ATOMS_PER_CRYSTAL, ATOMS_PER_CRYSTAL)] = (
            jnp.full((1, ATOMS_PER_CRYSTAL), stc_ref[cr0 + c], jnp.float32))


def _x_part(g, x_ref, nx_ref, sx_row):
    shp = (3, A_PER_BLK)
    f = (lax.broadcasted_iota(jnp.uint32, shp, 0)
         + jnp.uint32(3) * lax.broadcasted_iota(jnp.uint32, shp, 1)
         + jnp.uint32(3 * A_PER_BLK) * g)
    fu2 = _unit_float(_tf_bits(_K3, f))
    u2 = jnp.maximum(fu2 * _SPAN2 + _NEXT_M1, _NEXT_M1)
    nrm = _SQRT2 * _erfinv(u2)
    nx_ref[...] = x_ref[...] + nrm * sx_row[...]


def _tc_full_body(probsT_ref, h_ref, x_ref, stc_ref, sxc_ref,
                  nx_ref, ty_ref, st_row, sx_row):
    gi = pl.program_id(0) + C_SC // CR_PER_BLK
    g = lax.convert_element_type(gi, jnp.uint32)
    cr0 = gi * CR_PER_BLK
    _sigma_row(st_row, stc_ref, cr0)
    _sigma_row(sx_row, sxc_ref, cr0)

    shp = (MAX_ATOMIC_NUM, A_PER_BLK)
    f = (lax.broadcasted_iota(jnp.uint32, shp, 0)
         + jnp.uint32(100) * lax.broadcasted_iota(jnp.uint32, shp, 1)
         + jnp.uint32(100 * A_PER_BLK) * g)
    fu = _unit_float(_tf_bits(_K4, f))
    u = jnp.maximum(fu * _SPAN1 + _TINY, _TINY)
    gum = -jnp.log(-jnp.log(u))

    h = h_ref[...].reshape(1, A_PER_BLK)
    cio = lax.broadcasted_iota(jnp.int32, shp, 0)
    onehot = jnp.where(cio == h - jnp.int32(1), jnp.float32(1.0),
                       jnp.float32(0.0))
    val = gum + jnp.log(onehot + probsT_ref[...] * st_row[...])
    m = jnp.max(val, axis=0, keepdims=True)
    idx = jnp.min(jnp.where(val == m, cio, jnp.int32(MAX_ATOMIC_NUM)),
                  axis=0, keepdims=True)
    ty_ref[...] = (idx + jnp.int32(1)).reshape(1, 1, A_PER_BLK)

    _x_part(g, x_ref, nx_ref, sx_row)


def _tc_consume_body(probs_ref, h_ref, x_ref, stc_ref, sxc_ref, fu_ref,
                     _nx_al, nx_ref, ty_ref, st_col, sx_row):
    gi = pl.program_id(0)
    g = lax.convert_element_type(gi, jnp.uint32)
    cr0 = gi * CR_PER_BLK
    for c in range(CR_PER_BLK):
        st_col[pl.ds(c * ATOMS_PER_CRYSTAL, ATOMS_PER_CRYSTAL), :] = (
            jnp.full((ATOMS_PER_CRYSTAL, 1), stc_ref[cr0 + c], jnp.float32))
    _sigma_row(sx_row, sxc_ref, cr0)

    shp = (A_PER_BLK, MAX_ATOMIC_NUM)
    fu = fu_ref[...][:, :MAX_ATOMIC_NUM]
    u = jnp.maximum(fu * _SPAN1 + _TINY, _TINY)
    gum = -jnp.log(-jnp.log(u))

    cio = lax.broadcasted_iota(jnp.int32, shp, 1)
    onehot = jnp.where(cio == h_ref[...] - jnp.int32(1), jnp.float32(1.0),
                       jnp.float32(0.0))
    val = gum + jnp.log(onehot + probs_ref[...] * st_col[...])
    m = jnp.max(val, axis=1, keepdims=True)
    idx = jnp.min(jnp.where(val == m, cio, jnp.int32(MAX_ATOMIC_NUM)),
                  axis=1, keepdims=True)
    ty_ref[...] = idx + jnp.int32(1)

    _x_part(g, x_ref, nx_ref, sx_row)


def kernel(x, h, composition_probs, num_atoms, sigmas, type_sigmas):
    del num_atoms

    sxc, stc = pl.pallas_call(
        _levels_kernel,
        in_specs=[
            pl.BlockSpec(memory_space=pltpu.SMEM),
            pl.BlockSpec(memory_space=pltpu.SMEM),
        ],
        out_specs=[pl.BlockSpec((32, 128), lambda: (0, 0))] * 2,
        out_shape=[jax.ShapeDtypeStruct((32, 128), jnp.float32)] * 2,
    )(sigmas, type_sigmas)

    stcf = stc.reshape(N_CRYSTALS)

    sc_uniform = pl.kernel(
        _sc_uniform_kernel,
        out_type=jax.ShapeDtypeStruct((A_SC * 128,), jnp.float32),
        mesh=plsc.VectorSubcoreMesh(core_axis_name="c", subcore_axis_name="s"),
        scratch_types=[
            pltpu.VMEM((SC_CHUNK,), jnp.float32),
            pltpu.VMEM((SC_CHUNK,), jnp.float32),
            pltpu.SemaphoreType.DMA,
            pltpu.SemaphoreType.DMA,
        ],
    )
    fu2 = sc_uniform().reshape(A_SC, 128)

    sxcf = sxc.reshape(N_CRYSTALS)
    probsT = composition_probs.T
    xT = x.T
    hL = h.reshape(N_ATOMS // A_PER_BLK, 1, A_PER_BLK)

    n_full = N_CRYSTALS - C_SC
    b = C_SC // CR_PER_BLK
    nxT_b, ty_b = pl.pallas_call(
        _tc_full_body,
        grid=(n_full // CR_PER_BLK,),
        in_specs=[
            pl.BlockSpec((MAX_ATOMIC_NUM, A_PER_BLK), lambda g: (0, g + b)),
            pl.BlockSpec((1, 1, A_PER_BLK), lambda g: (g + b, 0, 0)),
            pl.BlockSpec((3, A_PER_BLK), lambda g: (0, g + b)),
            pl.BlockSpec(memory_space=pltpu.SMEM),
            pl.BlockSpec(memory_space=pltpu.SMEM),
        ],
        out_specs=[
            pl.BlockSpec((3, A_PER_BLK), lambda g: (0, g + b)),
            pl.BlockSpec((1, 1, A_PER_BLK), lambda g: (g + b, 0, 0)),
        ],
        out_shape=[
            jax.ShapeDtypeStruct((3, N_ATOMS), jnp.float32),
            jax.ShapeDtypeStruct((N_ATOMS // A_PER_BLK, 1, A_PER_BLK),
                                 jnp.int32),
        ],
        scratch_shapes=[pltpu.VMEM((1, A_PER_BLK), jnp.float32)] * 2,
    )(probsT, hL, xT, stcf, sxcf)

    probsA = composition_probs[:A_SC]
    hA = h[:A_SC].reshape(A_SC, 1)
    nxT, ty_a = pl.pallas_call(
        _tc_consume_body,
        grid=(C_SC // CR_PER_BLK,),
        in_specs=[
            pl.BlockSpec((A_PER_BLK, MAX_ATOMIC_NUM), lambda g: (g, 0)),
            pl.BlockSpec((A_PER_BLK, 1), lambda g: (g, 0)),
            pl.BlockSpec((3, A_PER_BLK), lambda g: (0, g)),
            pl.BlockSpec(memory_space=pltpu.SMEM),
            pl.BlockSpec(memory_space=pltpu.SMEM),
            pl.BlockSpec((A_PER_BLK, 128), lambda g: (g, 0)),
            pl.BlockSpec(memory_space=pl.ANY),
        ],
        out_specs=[
            pl.BlockSpec((3, A_PER_BLK), lambda g: (0, g)),
            pl.BlockSpec((A_PER_BLK, 1), lambda g: (g, 0)),
        ],
        out_shape=[
            jax.ShapeDtypeStruct((3, N_ATOMS), jnp.float32),
            jax.ShapeDtypeStruct((A_SC, 1), jnp.int32),
        ],
        scratch_shapes=[pltpu.VMEM((A_PER_BLK, 1), jnp.float32),
                        pltpu.VMEM((1, A_PER_BLK), jnp.float32)],
        input_output_aliases={6: 0},
    )(probsA, hA, xT, stcf, sxcf, fu2, nxT_b)

    ty = jnp.concatenate(
        [ty_a.reshape(A_SC), ty_b[b:].reshape(N_ATOMS - A_SC)])
    return nxT.T, ty


# speedup vs baseline: 1.0233x; 1.0233x over previous
import numpy as np
import jax
import jax.numpy as jnp
from jax import lax
from jax.experimental import pallas as pl
from jax.experimental.pallas import tpu as pltpu
from jax.experimental.pallas import tpu_sc as plsc

MAX_ATOMIC_NUM = 100
NUM_NOISE_LEVEL = 50
N_CRYSTALS = 4096
ATOMS_PER_CRYSTAL = 32
N_ATOMS = N_CRYSTALS * ATOMS_PER_CRYSTAL

CR_PER_BLK = 128
A_PER_BLK = CR_PER_BLK * ATOMS_PER_CRYSTAL

C_SC = 1536
A_SC = C_SC * ATOMS_PER_CRYSTAL
NW = 32
W_PER = A_SC * 128 // NW
SC_CHUNK = 16384
SC_NCHUNK = W_PER // SC_CHUNK

_U32 = np.uint32


def _np_threefry2x32(k0, k1, x0, x1):
    ks = [_U32(k0), _U32(k1), _U32(k0) ^ _U32(k1) ^ _U32(0x1BD11BDA)]
    rot = ([13, 15, 26, 6], [17, 29, 16, 24])
    x0 = np.asarray(x0, _U32)
    x1 = np.asarray(x1, _U32)
    with np.errstate(over="ignore"):
        x0 = x0 + ks[0]
        x1 = x1 + ks[1]
        for i in range(5):
            for r in rot[i % 2]:
                x0 = x0 + x1
                x1 = ((x1 << _U32(r)) | (x1 >> _U32(32 - r))) ^ x0
            x0 = x0 + ks[(i + 1) % 3]
            x1 = x1 + ks[(i + 2) % 3] + _U32(i + 1)
    return x0, x1


def _np_split(key, num):
    o0, o1 = _np_threefry2x32(key[0], key[1], np.zeros(num, _U32),
                              np.arange(num, dtype=_U32))
    return [(int(o0[i]), int(o1[i])) for i in range(num)]


_K1, _K2, _K3, _K4 = _np_split((0, 42), 4)
_K1A, _K1B = _np_split(_K1, 2)
_K2A, _K2B = _np_split(_K2, 2)

_TINY = np.float32(np.finfo(np.float32).tiny)
_NEXT_M1 = np.float32(np.nextafter(np.float32(-1.0), np.float32(0.0)))
_SPAN2 = np.float32(np.float32(1.0) - _NEXT_M1)
_SPAN1 = np.float32(np.float32(1.0) - _TINY)
_SQRT2 = np.float32(np.sqrt(2.0))


def _tf_bits(key, x1):
    k0, k1 = key
    ks = [k0 & 0xFFFFFFFF, k1 & 0xFFFFFFFF, (k0 ^ k1 ^ 0x1BD11BDA) & 0xFFFFFFFF]
    rot = ([13, 15, 26, 6], [17, 29, 16, 24])
    x0 = jnp.uint32(ks[0])
    x1 = x1 + jnp.uint32(ks[1])
    for i in range(5):
        for r in rot[i % 2]:
            x0 = x0 + x1
            x1 = lax.shift_left(x1, jnp.uint32(r)) | lax.shift_right_logical(
                x1, jnp.uint32(32 - r))
            x1 = x0 ^ x1
        x0 = x0 + jnp.uint32(ks[(i + 1) % 3])
        x1 = x1 + jnp.uint32((ks[(i + 2) % 3] + i + 1) & 0xFFFFFFFF)
    return x0 ^ x1


def _unit_float(bits):
    fb = lax.shift_right_logical(bits, jnp.uint32(9)) | jnp.uint32(0x3F800000)
    return lax.bitcast_convert_type(fb, jnp.float32) - jnp.float32(1.0)


def _mod50(u):
    a = lax.shift_right_logical(u, jnp.uint32(16))
    b = u & jnp.uint32(0xFFFF)
    m = (a * jnp.uint32(36) + b).astype(jnp.int32)
    q = jnp.floor(m.astype(jnp.float32) * jnp.float32(0.02)).astype(jnp.int32)
    r = m - q * jnp.int32(50)
    r = jnp.where(r < 0, r + jnp.int32(50), r)
    r = jnp.where(r >= jnp.int32(50), r - jnp.int32(50), r)
    r = jnp.where(r >= jnp.int32(50), r - jnp.int32(50), r)
    return r


def _erfinv(x):
    w = -jnp.log((jnp.float32(1.0) - x) * (jnp.float32(1.0) + x))
    ws = w - jnp.float32(2.5)
    wb = jnp.sqrt(w) - jnp.float32(3.0)
    cs = [2.81022636e-08, 3.43273939e-07, -3.5233877e-06, -4.39150654e-06,
          0.00021858087, -0.00125372503, -0.00417768164, 0.246640727,
          1.50140941]
    cb = [-0.000200214257, 0.000100950558, 0.00134934322, -0.00367342844,
          0.00573950773, -0.0076224613, 0.00943887047, 1.00167406, 2.83297682]
    ps = jnp.float32(cs[0])
    for c in cs[1:]:
        ps = jnp.float32(c) + ps * ws
    pb = jnp.float32(cb[0])
    for c in cb[1:]:
        pb = jnp.float32(c) + pb * wb
    return jnp.where(w < jnp.float32(5.0), ps, pb) * x


def _levels_kernel(sig_ref, tsig_ref, sx_ref, st_ref):
    i = (jnp.uint32(128) * lax.broadcasted_iota(jnp.uint32, (32, 128), 0)
         + lax.broadcasted_iota(jnp.uint32, (32, 128), 1))
    mult = jnp.int32(46)

    def levels(ka, kb):
        hi = _mod50(_tf_bits(ka, i))
        lo = _mod50(_tf_bits(kb, i))
        return _mod50((hi * mult + lo).astype(jnp.uint32))

    lvl_x = levels(_K1A, _K1B)
    lvl_t = levels(_K2A, _K2B)
    sx = jnp.zeros((32, 128), jnp.float32)
    st = jnp.zeros((32, 128), jnp.float32)
    for l in range(NUM_NOISE_LEVEL):
        sx = jnp.where(lvl_x == jnp.int32(l), sig_ref[l], sx)
        st = jnp.where(lvl_t == jnp.int32(l), tsig_ref[l], st)
    sx_ref[...] = sx
    st_ref[...] = st


def _sc_uniform_kernel(out_hbm, buf0, buf1, sem0, sem1):
    wid = lax.axis_index("s") * 2 + lax.axis_index("c")
    base = wid * W_PER
    lane = lax.iota(jnp.uint32, 16)
    bufs = (buf0, buf1)
    sems = (sem0, sem1)
    copies = []
    for step in range(SC_NCHUNK):
        buf = bufs[step % 2]
        if step >= 2:
            copies[step - 2].wait()
        cbase = lax.convert_element_type(base + step * SC_CHUNK, jnp.uint32)

        def body(j, _, buf=buf, cbase=cbase):
            q = cbase + lax.convert_element_type(j * 16, jnp.uint32) + lane
            c = (jnp.uint32(100) * lax.shift_right_logical(q, jnp.uint32(7))
                 + (q & jnp.uint32(127)))
            buf[pl.ds(j * 16, 16)] = _unit_float(_tf_bits(_K4, c))
            return 0

        lax.fori_loop(0, SC_CHUNK // 16, body, 0)
        cp = pltpu.make_async_copy(
            buf, out_hbm.at[pl.ds(base + step * SC_CHUNK, SC_CHUNK)],
            sems[step % 2])
        cp.start()
        copies.append(cp)
    copies[SC_NCHUNK - 2].wait()
    copies[SC_NCHUNK - 1].wait()


def _sigma_row(scr, stc_ref, cr0):
    for c in range(CR_PER_BLK):
        scr[:, pl.ds(c * ATOMS_PER_CRYSTAL, ATOMS_PER_CRYSTAL)] = (
            jnp.full((1, ATOMS_PER_CRYSTAL), stc_ref[cr0 + c], jnp.float32))


def _x_part(g, x_ref, nx_ref, sx_row):
    shp = (3, A_PER_BLK)
    f = (lax.broadcasted_iota(jnp.uint32, shp, 0)
         + jnp.uint32(3) * lax.broadcasted_iota(jnp.uint32, shp, 1)
         + jnp.uint32(3 * A_PER_BLK) * g)
    fu2 = _unit_float(_tf_bits(_K3, f))
    u2 = jnp.maximum(fu2 * _SPAN2 + _NEXT_M1, _NEXT_M1)
    nrm = _SQRT2 * _erfinv(u2)
    nx_ref[...] = x_ref[...] + nrm * sx_row[...]


def _tc_full_body(probsT_ref, h_ref, x_ref, stc_ref, sxc_ref,
                  nx_ref, ty_ref, st_row, sx_row):
    gi = pl.program_id(0) + C_SC // CR_PER_BLK
    g = lax.convert_element_type(gi, jnp.uint32)
    cr0 = gi * CR_PER_BLK
    _sigma_row(st_row, stc_ref, cr0)
    _sigma_row(sx_row, sxc_ref, cr0)

    shp = (MAX_ATOMIC_NUM, A_PER_BLK)
    f = (lax.broadcasted_iota(jnp.uint32, shp, 0)
         + jnp.uint32(100) * lax.broadcasted_iota(jnp.uint32, shp, 1)
         + jnp.uint32(100 * A_PER_BLK) * g)
    fu = _unit_float(_tf_bits(_K4, f))
    u = jnp.maximum(fu * _SPAN1 + _TINY, _TINY)
    gum = -jnp.log(-jnp.log(u))

    h = h_ref[...].reshape(1, A_PER_BLK)
    cio = lax.broadcasted_iota(jnp.int32, shp, 0)
    onehot = jnp.where(cio == h - jnp.int32(1), jnp.float32(1.0),
                       jnp.float32(0.0))
    val = gum + jnp.log(onehot + probsT_ref[...] * st_row[...])
    m = jnp.max(val, axis=0, keepdims=True)
    idx = jnp.min(jnp.where(val == m, cio, jnp.int32(MAX_ATOMIC_NUM)),
                  axis=0, keepdims=True)
    ty_ref[...] = (idx + jnp.int32(1)).reshape(1, 1, A_PER_BLK)

    _x_part(g, x_ref, nx_ref, sx_row)


def _tc_consume_body(probs_ref, h_ref, x_ref, stc_ref, sxc_ref, fu_ref,
                     _nx_al, nx_ref, ty_ref, st_col, sx_row):
    gi = pl.program_id(0)
    g = lax.convert_element_type(gi, jnp.uint32)
    cr0 = gi * CR_PER_BLK
    for c in range(CR_PER_BLK):
        st_col[pl.ds(c * ATOMS_PER_CRYSTAL, ATOMS_PER_CRYSTAL), :] = (
            jnp.full((ATOMS_PER_CRYSTAL, 1), stc_ref[cr0 + c], jnp.float32))
    _sigma_row(sx_row, sxc_ref, cr0)

    shp = (A_PER_BLK, MAX_ATOMIC_NUM)
    fu = fu_ref[...][:, :MAX_ATOMIC_NUM]
    u = jnp.maximum(fu * _SPAN1 + _TINY, _TINY)
    gum = -jnp.log(-jnp.log(u))

    cio = lax.broadcasted_iota(jnp.int32, shp, 1)
    onehot = jnp.where(cio == h_ref[...] - jnp.int32(1), jnp.float32(1.0),
                       jnp.float32(0.0))
    val = gum + jnp.log(onehot + probs_ref[...] * st_col[...])
    m = jnp.max(val, axis=1, keepdims=True)
    idx = jnp.min(jnp.where(val == m, cio, jnp.int32(MAX_ATOMIC_NUM)),
                  axis=1, keepdims=True)
    ty_ref[...] = idx + jnp.int32(1)

    _x_part(g, x_ref, nx_ref, sx_row)


def kernel(x, h, composition_probs, num_atoms, sigmas, type_sigmas):
    del num_atoms

    sxc, stc = pl.pallas_call(
        _levels_kernel,
        in_specs=[
            pl.BlockSpec(memory_space=pltpu.SMEM),
            pl.BlockSpec(memory_space=pltpu.SMEM),
        ],
        out_specs=[pl.BlockSpec((32, 128), lambda: (0, 0))] * 2,
        out_shape=[jax.ShapeDtypeStruct((32, 128), jnp.float32)] * 2,
    )(sigmas, type_sigmas)

    stcf = stc.reshape(N_CRYSTALS)

    sc_uniform = pl.kernel(
        _sc_uniform_kernel,
        out_type=jax.ShapeDtypeStruct((A_SC * 128,), jnp.float32),
        mesh=plsc.VectorSubcoreMesh(core_axis_name="c", subcore_axis_name="s"),
        scratch_types=[
            pltpu.VMEM((SC_CHUNK,), jnp.float32),
            pltpu.VMEM((SC_CHUNK,), jnp.float32),
            pltpu.SemaphoreType.DMA,
            pltpu.SemaphoreType.DMA,
        ],
    )
    fu2 = sc_uniform().reshape(A_SC, 128)

    sxcf = sxc.reshape(N_CRYSTALS)
    probsT = composition_probs.T
    xT = x.T
    hL = h.reshape(N_ATOMS // A_PER_BLK, 1, A_PER_BLK)

    n_full = N_CRYSTALS - C_SC
    b = C_SC // CR_PER_BLK
    nxT_b, ty_b = pl.pallas_call(
        _tc_full_body,
        grid=(n_full // CR_PER_BLK,),
        in_specs=[
            pl.BlockSpec((MAX_ATOMIC_NUM, A_PER_BLK), lambda g: (0, g + b)),
            pl.BlockSpec((1, 1, A_PER_BLK), lambda g: (g + b, 0, 0)),
            pl.BlockSpec((3, A_PER_BLK), lambda g: (0, g + b)),
            pl.BlockSpec(memory_space=pltpu.SMEM),
            pl.BlockSpec(memory_space=pltpu.SMEM),
        ],
        out_specs=[
            pl.BlockSpec((3, A_PER_BLK), lambda g: (0, g + b)),
            pl.BlockSpec((1, 1, A_PER_BLK), lambda g: (g + b, 0, 0)),
        ],
        out_shape=[
            jax.ShapeDtypeStruct((3, N_ATOMS), jnp.float32),
            jax.ShapeDtypeStruct((N_ATOMS // A_PER_BLK, 1, A_PER_BLK),
                                 jnp.int32),
        ],
        scratch_shapes=[pltpu.VMEM((1, A_PER_BLK), jnp.float32)] * 2,
    )(probsT, hL, xT, stcf, sxcf)

    probsA = composition_probs[:A_SC]
    hA = h[:A_SC].reshape(A_SC, 1)
    nxT, ty_a = pl.pallas_call(
        _tc_consume_body,
        grid=(C_SC // CR_PER_BLK,),
        in_specs=[
            pl.BlockSpec((A_PER_BLK, MAX_ATOMIC_NUM), lambda g: (g, 0)),
            pl.BlockSpec((A_PER_BLK, 1), lambda g: (g, 0)),
            pl.BlockSpec((3, A_PER_BLK), lambda g: (0, g)),
            pl.BlockSpec(memory_space=pltpu.SMEM),
            pl.BlockSpec(memory_space=pltpu.SMEM),
            pl.BlockSpec((A_PER_BLK, 128), lambda g: (g, 0)),
            pl.BlockSpec(memory_space=pl.ANY),
        ],
        out_specs=[
            pl.BlockSpec((3, A_PER_BLK), lambda g: (0, g)),
            pl.BlockSpec((A_PER_BLK, 1), lambda g: (g, 0)),
        ],
        out_shape=[
            jax.ShapeDtypeStruct((3, N_ATOMS), jnp.float32),
            jax.ShapeDtypeStruct((A_SC, 1), jnp.int32),
        ],
        scratch_shapes=[pltpu.VMEM((A_PER_BLK, 1), jnp.float32),
                        pltpu.VMEM((1, A_PER_BLK), jnp.float32)],
        input_output_aliases={6: 0},
    )(probsA, hA, xT, stcf, sxcf, fu2, nxT_b)

    ty = jnp.concatenate(
        [ty_a.reshape(A_SC), ty_b[b:].reshape(N_ATOMS - A_SC)])
    return nxT.T, ty
